# baseline (speedup 1.0000x reference)
import jax
import jax.numpy as jnp
from jax.experimental import pallas as pl
from jax.experimental.pallas import tpu as pltpu

N = 10000
BMAX = 256
SIZES = [64, 128] + [BMAX] * 38 + [80]
assert sum(SIZES) == N
OFFS = [sum(SIZES[:j]) for j in range(len(SIZES))]
NBUF = 4
Q = 4


def _gcn_kernel(x_hbm, adj_hbm, w_ref, b_ref, out_ref, xv_ref, sup_ref,
                buf_ref, sem, xsem):
    def copies(step):
        sz, off = SIZES[step], OFFS[step]
        nq = Q if sz % (8 * Q) == 0 else (2 if sz % 16 == 0 else 1)
        sub = sz // nq
        return [
            pltpu.make_async_copy(
                adj_hbm.at[pl.ds(off + q * sub, sub), :],
                buf_ref.at[step % NBUF, pl.ds(q * sub, sub), :],
                sem.at[step % NBUF, q],
            )
            for q in range(nq)
        ]

    xcopy = pltpu.make_async_copy(x_hbm, xv_ref, xsem)
    xcopy.start()
    for j in range(NBUF - 1):
        for c in copies(j):
            c.start()
    xcopy.wait()
    sup_ref[:, :] = jnp.dot(
        xv_ref[:, :], w_ref[:, :], preferred_element_type=jnp.float32
    )

    for step, (sz, off) in enumerate(zip(SIZES, OFFS)):
        for c in copies(step):
            c.wait()
        if step + NBUF - 1 < len(SIZES):
            for c in copies(step + NBUF - 1):
                c.start()
        h = jnp.dot(
            buf_ref[step % NBUF, 0:sz, :],
            sup_ref[:, :],
            preferred_element_type=jnp.float32,
        )
        h = jax.nn.relu(h + b_ref[:, :])
        m = jnp.max(h, axis=1, keepdims=True)
        lse = jnp.log(jnp.sum(jnp.exp(h - m), axis=1, keepdims=True)) + m
        out_ref[pl.ds(off, sz), :] = h - lse


@jax.jit
def _run(x, adj, W, b):
    nhid = W.shape[1]
    nfeat = x.shape[1]
    return pl.pallas_call(
        _gcn_kernel,
        in_specs=[
            pl.BlockSpec(memory_space=pl.ANY),
            pl.BlockSpec(memory_space=pl.ANY),
            pl.BlockSpec(memory_space=pltpu.VMEM),
            pl.BlockSpec(memory_space=pltpu.VMEM),
        ],
        out_specs=pl.BlockSpec(memory_space=pltpu.VMEM),
        out_shape=jax.ShapeDtypeStruct((N, nhid), jnp.float32),
        scratch_shapes=[
            pltpu.VMEM((N, nfeat), jnp.float32),
            pltpu.VMEM((N, nhid), jnp.float32),
            pltpu.VMEM((NBUF, BMAX, N), jnp.float32),
            pltpu.SemaphoreType.DMA((NBUF, Q)),
            pltpu.SemaphoreType.DMA,
        ],
        compiler_params=pltpu.CompilerParams(
            vmem_limit_bytes=100 * 1024 * 1024,
        ),
    )(x, adj, W, b)


def kernel(x, adj, W, b):
    return _run(x, adj, W, b.reshape(1, -1))


# speedup vs baseline: 1.0218x; 1.0218x over previous
import jax
import jax.numpy as jnp
from jax.experimental import pallas as pl
from jax.experimental.pallas import tpu as pltpu

N = 10000
NHID = 16
BM = 200
NM = N // BM
NBUF = 4


def _gcn_kernel(x_ref, adj_hbm, w_ref, b_ref, out_hbm, sup_ref):
    sup_ref[:, :] = jnp.dot(
        x_ref[:, :], w_ref[:, :], preferred_element_type=jnp.float32
    )

    def inner(adj_ref, out_ref):
        h = jnp.dot(
            adj_ref[:, :], sup_ref[:, :], preferred_element_type=jnp.float32
        )
        h = jax.nn.relu(h + b_ref[:, :])
        m = jnp.max(h, axis=1, keepdims=True)
        lse = jnp.log(jnp.sum(jnp.exp(h - m), axis=1, keepdims=True)) + m
        out_ref[:, :] = h - lse

    pltpu.emit_pipeline(
        inner,
        grid=(NM,),
        in_specs=[
            pl.BlockSpec(
                (BM, N),
                lambda i: (i, 0),
                pipeline_mode=pl.Buffered(buffer_count=NBUF),
            ),
        ],
        out_specs=[pl.BlockSpec((BM, NHID), lambda i: (i, 0))],
    )(adj_hbm, out_hbm)


@jax.jit
def _run(x, adj, W, b):
    return pl.pallas_call(
        _gcn_kernel,
        in_specs=[
            pl.BlockSpec(memory_space=pltpu.VMEM),
            pl.BlockSpec(memory_space=pl.ANY),
            pl.BlockSpec(memory_space=pltpu.VMEM),
            pl.BlockSpec(memory_space=pltpu.VMEM),
        ],
        out_specs=pl.BlockSpec(memory_space=pl.ANY),
        out_shape=jax.ShapeDtypeStruct((N, NHID), jnp.float32),
        scratch_shapes=[
            pltpu.VMEM((N, NHID), jnp.float32),
        ],
        compiler_params=pltpu.CompilerParams(
            vmem_limit_bytes=100 * 1024 * 1024,
        ),
    )(x, adj, W, b)


def kernel(x, adj, W, b):
    return _run(x, adj, W, b.reshape(1, -1))
